# Initial kernel scaffold; baseline (speedup 1.0000x reference)
import jax
import jax.numpy as jnp
from jax.experimental import pallas as pl


def kernel(x, edge_index, Wl1, Wr1, att1, b1, Wl2, Wr2, att2, b2):
    raise NotImplementedError("write your pallas kernel here")


# speedup vs baseline: 11.3334x; 11.3334x over previous
import functools

import jax
import jax.numpy as jnp
from jax import lax
from jax.experimental import pallas as pl
from jax.experimental.pallas import tpu as pltpu
from jax.experimental.pallas import tpu_sc as plsc

N = 10000
E = 320000
N_PAD = 10240
NUM_TILES = 32
EDGES_PER_TILE = E // NUM_TILES
CHUNK = 80
NCHUNK = EDGES_PER_TILE // CHUNK
GROUPS = CHUNK // 16


def _sc_edge_pass(H, C, W):
    HC = H * C
    mesh = plsc.VectorSubcoreMesh(core_axis_name="c", subcore_axis_name="s")

    @functools.partial(
        pl.kernel,
        mesh=mesh,
        out_type=jax.ShapeDtypeStruct((2 * N_PAD, W), jnp.float32),
        compiler_params=pltpu.CompilerParams(needs_layout_passes=False,
                                             use_tc_tiling_on_sc=False),
        scratch_types=[
            pltpu.VMEM((HC + 16,), jnp.float32),
            pltpu.VMEM((CHUNK,), jnp.int32),
            pltpu.VMEM((CHUNK,), jnp.int32),
            pltpu.VMEM((CHUNK, HC), jnp.float32),
            pltpu.VMEM((CHUNK, HC), jnp.float32),
            pltpu.VMEM((CHUNK, W), jnp.float32),
            pltpu.VMEM_SHARED((N_PAD, W), jnp.float32),
            pltpu.SemaphoreType.DMA,
            pltpu.SemaphoreType.DMA,
        ],
    )
    def edge_pass(xl_hbm, xr_hbm, att_hbm, src_hbm, dst_hbm, out_hbm,
                  att_v, src_v, dst_v, xl_buf, xr_buf, w_buf, num_shared,
                  sem1, sem2):
        cid = lax.axis_index("c")
        sid = lax.axis_index("s")
        wid = cid * 16 + sid
        edge_base = wid * EDGES_PER_TILE

        pltpu.sync_copy(att_hbm, att_v)

        def zero_body(j, carry):
            zi = (j // (W // 16)).astype(jnp.int32)
            zj = (j % (W // 16)).astype(jnp.int32)
            w_buf[zi, pl.ds(zj * 16, 16)] = jnp.zeros((16,), jnp.float32)
            return carry
        lax.fori_loop(0, CHUNK * (W // 16), zero_body, 0)
        rows_per_tile = N_PAD // 16
        for k in range(rows_per_tile // CHUNK):
            pltpu.sync_copy(
                w_buf, num_shared.at[pl.ds(sid * rows_per_tile + k * CHUNK,
                                           CHUNK)])
        plsc.subcore_barrier()

        lane = lax.iota(jnp.int32, 16)

        def chunk_body(i, carry):
            base = edge_base + i * CHUNK
            pltpu.sync_copy(src_hbm.at[pl.ds(base, CHUNK)], src_v)
            pltpu.sync_copy(dst_hbm.at[pl.ds(base, CHUNK)], dst_v)
            c1 = pltpu.async_copy(xl_hbm.at[src_v], xl_buf, sem1)
            c2 = pltpu.async_copy(xr_hbm.at[dst_v], xr_buf, sem2)
            c1.wait()
            c2.wait()

            def group_body(g, gcarry):
                eidx = g * 16 + lane
                for h in range(H):
                    def alpha_body(c2i, acc):
                        c = h * C + c2i
                        cvec = jnp.full((16,), c, jnp.int32)
                        xlc = plsc.load_gather(xl_buf, [eidx, cvec])
                        xrc = plsc.load_gather(xr_buf, [eidx, cvec])
                        m = xlc + xrc
                        m = jnp.maximum(m, 0.2 * m)
                        return acc + m * att_v[pl.ds(c, 16)][0]
                    acc = lax.fori_loop(0, C, alpha_body,
                                        jnp.zeros((16,), jnp.float32))
                    ex = jnp.exp(acc)
                    plsc.store_scatter(
                        w_buf, [eidx, jnp.full((16,), HC + h, jnp.int32)], ex)

                    def weight_body(c2i, wcarry):
                        c = h * C + c2i
                        cvec = jnp.full((16,), c, jnp.int32)
                        xlc = plsc.load_gather(xl_buf, [eidx, cvec])
                        plsc.store_scatter(w_buf, [eidx, cvec], xlc * ex)
                        return wcarry
                    lax.fori_loop(0, C, weight_body, 0)
                return gcarry
            lax.fori_loop(0, GROUPS, group_body, 0)

            pltpu.sync_copy(w_buf, num_shared.at[dst_v], add=True)
            return carry
        lax.fori_loop(0, NCHUNK, chunk_body, 0)

        plsc.subcore_barrier()
        off = cid * N_PAD + sid * rows_per_tile
        pltpu.sync_copy(num_shared.at[pl.ds(sid * rows_per_tile,
                                            rows_per_tile)],
                        out_hbm.at[pl.ds(off, rows_per_tile)])

    return edge_pass


_sc_layer1 = _sc_edge_pass(4, 32, 144)
_sc_layer2 = _sc_edge_pass(1, 16, 32)


def _mm2_body(x_ref, wa_ref, wb_ref, oa_ref, ob_ref):
    x = x_ref[...]
    oa_ref[...] = jnp.dot(x, wa_ref[...], preferred_element_type=jnp.float32)
    ob_ref[...] = jnp.dot(x, wb_ref[...], preferred_element_type=jnp.float32)


def _mm2(x, wa_t, wb_t):
    n, d = x.shape
    k = wa_t.shape[1]
    bn = 1000 if n == N else 1024
    return pl.pallas_call(
        _mm2_body,
        grid=(n // bn,),
        in_specs=[
            pl.BlockSpec((bn, d), lambda i: (i, 0)),
            pl.BlockSpec((d, k), lambda i: (0, 0)),
            pl.BlockSpec((d, k), lambda i: (0, 0)),
        ],
        out_specs=[
            pl.BlockSpec((bn, k), lambda i: (i, 0)),
            pl.BlockSpec((bn, k), lambda i: (i, 0)),
        ],
        out_shape=[
            jax.ShapeDtypeStruct((n, k), jnp.float32),
            jax.ShapeDtypeStruct((n, k), jnp.float32),
        ],
    )(x, wa_t, wb_t)


def _combine1_body(s0_ref, s1_ref, sel_ref, b1_ref, wl_ref, wr_ref,
                   xl_ref, xr_ref):
    s0 = s0_ref[...]
    s1 = s1_ref[...]
    num = s0[:, :128] + s1[:, :128]
    den4 = s0[:, 128:132] + s1[:, 128:132]
    den = jnp.dot(den4, sel_ref[...], preferred_element_type=jnp.float32)
    h = jnp.maximum(num / (den + 1e-16) + b1_ref[...], 0.0)
    xl_ref[...] = jnp.dot(h, wl_ref[...], preferred_element_type=jnp.float32)
    xr_ref[...] = jnp.dot(h, wr_ref[...], preferred_element_type=jnp.float32)


def _combine1(s, sel, b1, wl2_t, wr2_t):
    bn = 1024
    k = wl2_t.shape[1]
    return pl.pallas_call(
        _combine1_body,
        grid=(N_PAD // bn,),
        in_specs=[
            pl.BlockSpec((bn, 144), lambda i: (i, 0)),
            pl.BlockSpec((bn, 144), lambda i: (i + N_PAD // bn, 0)),
            pl.BlockSpec((4, 128), lambda i: (0, 0)),
            pl.BlockSpec((1, 128), lambda i: (0, 0)),
            pl.BlockSpec((128, k), lambda i: (0, 0)),
            pl.BlockSpec((128, k), lambda i: (0, 0)),
        ],
        out_specs=[
            pl.BlockSpec((bn, k), lambda i: (i, 0)),
            pl.BlockSpec((bn, k), lambda i: (i, 0)),
        ],
        out_shape=[
            jax.ShapeDtypeStruct((N_PAD, k), jnp.float32),
            jax.ShapeDtypeStruct((N_PAD, k), jnp.float32),
        ],
    )(s, s, sel, b1.reshape(1, 128), wl2_t, wr2_t)


def _combine2_body(s0_ref, s1_ref, b2_ref, o_ref):
    s0 = s0_ref[...]
    s1 = s1_ref[...]
    num = s0[:, :16] + s1[:, :16]
    den = s0[:, 16:17] + s1[:, 16:17]
    o_ref[...] = num / (den + 1e-16) + b2_ref[...]


def _combine2(s, b2):
    bn = 1024
    return pl.pallas_call(
        _combine2_body,
        grid=(N_PAD // bn,),
        in_specs=[
            pl.BlockSpec((bn, 32), lambda i: (i, 0)),
            pl.BlockSpec((bn, 32), lambda i: (i + N_PAD // bn, 0)),
            pl.BlockSpec((1, 16), lambda i: (0, 0)),
        ],
        out_specs=pl.BlockSpec((bn, 16), lambda i: (i, 0)),
        out_shape=jax.ShapeDtypeStruct((N_PAD, 16), jnp.float32),
    )(s, s, b2.reshape(1, 16))


def kernel(x, edge_index, Wl1, Wr1, att1, b1, Wl2, Wr2, att2, b2):
    src = edge_index[0]
    dst = edge_index[1]
    sel = jnp.repeat(jnp.eye(4, dtype=jnp.float32), 32, axis=1)

    att1f = jnp.pad(att1.reshape(-1), (0, 16))
    att2f = jnp.pad(att2.reshape(-1), (0, 16))

    xl1, xr1 = _mm2(x, Wl1.T, Wr1.T)
    s1 = _sc_layer1(xl1, xr1, att1f, src, dst)
    xl2, xr2 = _combine1(s1, sel, b1, Wl2.T, Wr2.T)
    s2 = _sc_layer2(xl2, xr2, att2f, src, dst)
    out = _combine2(s2, b2)
    return out[:N]


# speedup vs baseline: 11.3345x; 1.0001x over previous
import functools

import jax
import jax.numpy as jnp
from jax import lax
from jax.experimental import pallas as pl
from jax.experimental.pallas import tpu as pltpu
from jax.experimental.pallas import tpu_sc as plsc

N = 10000
E = 320000
N_PAD = 10240
NUM_TILES = 32
EDGES_PER_TILE = E // NUM_TILES
CHUNK = 16
NCHUNK = EDGES_PER_TILE // CHUNK
GROUPS = CHUNK // 16


def _sc_edge_pass(H, C, W):
    HC = H * C
    mesh = plsc.VectorSubcoreMesh(core_axis_name="c", subcore_axis_name="s")

    @functools.partial(
        pl.kernel,
        mesh=mesh,
        out_type=jax.ShapeDtypeStruct((2 * N_PAD, W), jnp.float32),
        compiler_params=pltpu.CompilerParams(needs_layout_passes=False,
                                             use_tc_tiling_on_sc=False),
        scratch_types=[
            pltpu.VMEM((HC + 16,), jnp.float32),
            pltpu.VMEM((2, CHUNK), jnp.int32),
            pltpu.VMEM((2, CHUNK), jnp.int32),
            pltpu.VMEM((2, CHUNK, HC), jnp.float32),
            pltpu.VMEM((2, CHUNK, HC), jnp.float32),
            pltpu.VMEM((2, CHUNK, W), jnp.float32),
            pltpu.VMEM_SHARED((N_PAD, W), jnp.float32),
            pltpu.SemaphoreType.DMA,
            pltpu.SemaphoreType.DMA,
            pltpu.SemaphoreType.DMA,
            pltpu.SemaphoreType.DMA,
            pltpu.SemaphoreType.DMA,
            pltpu.SemaphoreType.DMA,
        ],
    )
    def edge_pass(xl_hbm, xr_hbm, att_hbm, src_hbm, dst_hbm, out_hbm,
                  att_v, src_v, dst_v, xl_buf, xr_buf, w_buf, num_shared,
                  semi_a, semi_b, semxl_a, semxl_b, semxr_a, semxr_b):
        cid = lax.axis_index("c")
        sid = lax.axis_index("s")
        wid = cid * 16 + sid
        edge_base = wid * EDGES_PER_TILE
        semi = (semi_a, semi_b)
        semxl = (semxl_a, semxl_b)
        semxr = (semxr_a, semxr_b)

        pltpu.sync_copy(att_hbm, att_v)

        for b in range(2):
            def zero_body(j, carry, _b=b):
                zi = (j // (W // 16)).astype(jnp.int32)
                zj = (j % (W // 16)).astype(jnp.int32)
                w_buf[_b, zi, pl.ds(zj * 16, 16)] = jnp.zeros((16,),
                                                             jnp.float32)
                return carry
            lax.fori_loop(0, CHUNK * (W // 16), zero_body, 0)
        rows_per_tile = N_PAD // 16
        for k in range(rows_per_tile // CHUNK):
            pltpu.sync_copy(
                w_buf.at[0],
                num_shared.at[pl.ds(sid * rows_per_tile + k * CHUNK, CHUNK)])
        plsc.subcore_barrier()

        lane = lax.iota(jnp.int32, 16)

        def issue_idx(c, b):
            base = edge_base + c * CHUNK
            pltpu.async_copy(src_hbm.at[pl.ds(base, CHUNK)], src_v.at[b],
                             semi[b])
            pltpu.async_copy(dst_hbm.at[pl.ds(base, CHUNK)], dst_v.at[b],
                             semi[b])

        def wait_idx(b):
            pltpu.make_async_copy(src_hbm.at[pl.ds(0, CHUNK)], src_v.at[b],
                                  semi[b]).wait()
            pltpu.make_async_copy(dst_hbm.at[pl.ds(0, CHUNK)], dst_v.at[b],
                                  semi[b]).wait()

        def issue_gather(b):
            pltpu.async_copy(xl_hbm.at[src_v.at[b]], xl_buf.at[b], semxl[b])
            pltpu.async_copy(xr_hbm.at[dst_v.at[b]], xr_buf.at[b], semxr[b])

        def wait_gather(b):
            pltpu.make_async_copy(xl_hbm.at[src_v.at[b]], xl_buf.at[b],
                                  semxl[b]).wait()
            pltpu.make_async_copy(xr_hbm.at[dst_v.at[b]], xr_buf.at[b],
                                  semxr[b]).wait()

        def compute(b):
            xb = xl_buf.at[b]
            rb = xr_buf.at[b]
            wb = w_buf.at[b]

            def group_body(g, gcarry):
                eidx = g * 16 + lane
                for h in range(H):
                    acc = jnp.zeros((16,), jnp.float32)
                    for c2i in range(C):
                        c = h * C + c2i
                        cvec = jnp.full((16,), c, jnp.int32)
                        xlc = plsc.load_gather(xb, [eidx, cvec])
                        xrc = plsc.load_gather(rb, [eidx, cvec])
                        m = xlc + xrc
                        m = jnp.maximum(m, 0.2 * m)
                        acc = acc + m * att_v[pl.ds(c, 16)][0]
                    ex = jnp.exp(acc)
                    plsc.store_scatter(
                        wb, [eidx, jnp.full((16,), HC + h, jnp.int32)], ex)
                    for c2i in range(C):
                        c = h * C + c2i
                        cvec = jnp.full((16,), c, jnp.int32)
                        xlc = plsc.load_gather(xb, [eidx, cvec])
                        plsc.store_scatter(wb, [eidx, cvec], xlc * ex)
                return gcarry
            lax.fori_loop(0, GROUPS, group_body, 0)

        def scatter_out(b):
            pltpu.sync_copy(w_buf.at[b], num_shared.at[dst_v.at[b]],
                            add=True)

        def half_iter(c, b):
            issue_gather(1 - b)
            wait_gather(b)
            compute(b)
            scatter_out(b)
            @pl.when(c + 2 < NCHUNK)
            def _():
                issue_idx(c + 2, b)

        issue_idx(0, 0)
        wait_idx(0)
        issue_gather(0)
        issue_idx(1, 1)

        def pair_body(i, carry):
            c0 = 2 * i
            wait_idx(1)
            half_iter(c0, 0)
            wait_idx(0)
            half_iter(c0 + 1, 1)
            return carry
        lax.fori_loop(0, NCHUNK // 2, pair_body, 0)

        wait_gather(0)
        compute(0)
        scatter_out(0)

        plsc.subcore_barrier()
        off = cid * N_PAD + sid * rows_per_tile
        pltpu.sync_copy(num_shared.at[pl.ds(sid * rows_per_tile,
                                            rows_per_tile)],
                        out_hbm.at[pl.ds(off, rows_per_tile)])

    return edge_pass


_sc_layer1 = _sc_edge_pass(4, 32, 144)
_sc_layer2 = _sc_edge_pass(1, 16, 32)


def _mm2_body(x_ref, wa_ref, wb_ref, oa_ref, ob_ref):
    x = x_ref[...]
    oa_ref[...] = jnp.dot(x, wa_ref[...], preferred_element_type=jnp.float32)
    ob_ref[...] = jnp.dot(x, wb_ref[...], preferred_element_type=jnp.float32)


def _mm2(x, wa_t, wb_t):
    n, d = x.shape
    k = wa_t.shape[1]
    bn = 1000 if n == N else 1024
    return pl.pallas_call(
        _mm2_body,
        grid=(n // bn,),
        in_specs=[
            pl.BlockSpec((bn, d), lambda i: (i, 0)),
            pl.BlockSpec((d, k), lambda i: (0, 0)),
            pl.BlockSpec((d, k), lambda i: (0, 0)),
        ],
        out_specs=[
            pl.BlockSpec((bn, k), lambda i: (i, 0)),
            pl.BlockSpec((bn, k), lambda i: (i, 0)),
        ],
        out_shape=[
            jax.ShapeDtypeStruct((n, k), jnp.float32),
            jax.ShapeDtypeStruct((n, k), jnp.float32),
        ],
    )(x, wa_t, wb_t)


def _combine1_body(s0_ref, s1_ref, sel_ref, b1_ref, wl_ref, wr_ref,
                   xl_ref, xr_ref):
    s0 = s0_ref[...]
    s1 = s1_ref[...]
    num = s0[:, :128] + s1[:, :128]
    den4 = s0[:, 128:132] + s1[:, 128:132]
    den = jnp.dot(den4, sel_ref[...], preferred_element_type=jnp.float32)
    h = jnp.maximum(num / (den + 1e-16) + b1_ref[...], 0.0)
    xl_ref[...] = jnp.dot(h, wl_ref[...], preferred_element_type=jnp.float32)
    xr_ref[...] = jnp.dot(h, wr_ref[...], preferred_element_type=jnp.float32)


def _combine1(s, sel, b1, wl2_t, wr2_t):
    bn = 1024
    k = wl2_t.shape[1]
    return pl.pallas_call(
        _combine1_body,
        grid=(N_PAD // bn,),
        in_specs=[
            pl.BlockSpec((bn, 144), lambda i: (i, 0)),
            pl.BlockSpec((bn, 144), lambda i: (i + N_PAD // bn, 0)),
            pl.BlockSpec((4, 128), lambda i: (0, 0)),
            pl.BlockSpec((1, 128), lambda i: (0, 0)),
            pl.BlockSpec((128, k), lambda i: (0, 0)),
            pl.BlockSpec((128, k), lambda i: (0, 0)),
        ],
        out_specs=[
            pl.BlockSpec((bn, k), lambda i: (i, 0)),
            pl.BlockSpec((bn, k), lambda i: (i, 0)),
        ],
        out_shape=[
            jax.ShapeDtypeStruct((N_PAD, k), jnp.float32),
            jax.ShapeDtypeStruct((N_PAD, k), jnp.float32),
        ],
    )(s, s, sel, b1.reshape(1, 128), wl2_t, wr2_t)


def _combine2_body(s0_ref, s1_ref, b2_ref, o_ref):
    s0 = s0_ref[...]
    s1 = s1_ref[...]
    num = s0[:, :16] + s1[:, :16]
    den = s0[:, 16:17] + s1[:, 16:17]
    o_ref[...] = num / (den + 1e-16) + b2_ref[...]


def _combine2(s, b2):
    bn = 1024
    return pl.pallas_call(
        _combine2_body,
        grid=(N_PAD // bn,),
        in_specs=[
            pl.BlockSpec((bn, 32), lambda i: (i, 0)),
            pl.BlockSpec((bn, 32), lambda i: (i + N_PAD // bn, 0)),
            pl.BlockSpec((1, 16), lambda i: (0, 0)),
        ],
        out_specs=pl.BlockSpec((bn, 16), lambda i: (i, 0)),
        out_shape=jax.ShapeDtypeStruct((N_PAD, 16), jnp.float32),
    )(s, s, b2.reshape(1, 16))


def kernel(x, edge_index, Wl1, Wr1, att1, b1, Wl2, Wr2, att2, b2):
    src = edge_index[0]
    dst = edge_index[1]
    sel = jnp.repeat(jnp.eye(4, dtype=jnp.float32), 32, axis=1)
    att1f = jnp.pad(att1.reshape(-1), (0, 16))
    att2f = jnp.pad(att2.reshape(-1), (0, 16))

    xl1, xr1 = _mm2(x, Wl1.T, Wr1.T)
    s1 = _sc_layer1(xl1, xr1, att1f, src, dst)
    xl2, xr2 = _combine1(s1, sel, b1, Wl2.T, Wr2.T)
    s2 = _sc_layer2(xl2, xr2, att2f, src, dst)
    out = _combine2(s2, b2)
    return out[:N]


# speedup vs baseline: 19.6173x; 1.7308x over previous
import functools

import jax
import jax.numpy as jnp
from jax import lax
from jax.experimental import pallas as pl
from jax.experimental.pallas import tpu as pltpu
from jax.experimental.pallas import tpu_sc as plsc

N = 10000
E = 320000
N_PAD = 10240
NUM_TILES = 32
EDGES_PER_TILE = E // NUM_TILES
CHUNK = 16
NCHUNK = EDGES_PER_TILE // CHUNK
GROUPS = CHUNK // 16


def _sc_edge_pass(H, C, W):
    HC = H * C
    zero_starts = sorted(set(list(range(0, W - 15, 16)) + [W - 16]))
    mesh = plsc.VectorSubcoreMesh(core_axis_name="c", subcore_axis_name="s")

    @functools.partial(
        pl.kernel,
        mesh=mesh,
        out_type=jax.ShapeDtypeStruct((2 * N_PAD, W), jnp.float32),
        compiler_params=pltpu.CompilerParams(needs_layout_passes=False,
                                             use_tc_tiling_on_sc=False),
        scratch_types=[
            pltpu.VMEM((HC + 16,), jnp.float32),
            pltpu.VMEM((2, CHUNK), jnp.int32),
            pltpu.VMEM((2, CHUNK), jnp.int32),
            pltpu.VMEM((2, CHUNK, W), jnp.float32),
            pltpu.VMEM((2, CHUNK, W), jnp.float32),
            pltpu.VMEM((2, CHUNK, W), jnp.float32),
            pltpu.VMEM_SHARED((N_PAD, W), jnp.float32),
            pltpu.SemaphoreType.DMA,
            pltpu.SemaphoreType.DMA,
            pltpu.SemaphoreType.DMA,
            pltpu.SemaphoreType.DMA,
            pltpu.SemaphoreType.DMA,
            pltpu.SemaphoreType.DMA,
        ],
    )
    def edge_pass(xl_hbm, xr_hbm, att_hbm, src_hbm, dst_hbm, out_hbm,
                  att_v, src_v, dst_v, xl_buf, xr_buf, w_buf, num_shared,
                  semi_a, semi_b, semxl_a, semxl_b, semxr_a, semxr_b):
        cid = lax.axis_index("c")
        sid = lax.axis_index("s")
        wid = cid * 16 + sid
        edge_base = wid * EDGES_PER_TILE
        semi = (semi_a, semi_b)
        semxl = (semxl_a, semxl_b)
        semxr = (semxr_a, semxr_b)

        pltpu.sync_copy(att_hbm, att_v)

        for b in range(2):
            def zero_body(j, carry, _b=b):
                for st in zero_starts:
                    w_buf[_b, j, pl.ds(st, 16)] = jnp.zeros((16,),
                                                            jnp.float32)
                return carry
            lax.fori_loop(0, CHUNK, zero_body, 0)
        rows_per_tile = N_PAD // 16
        for k in range(rows_per_tile // CHUNK):
            pltpu.sync_copy(
                w_buf.at[0],
                num_shared.at[pl.ds(sid * rows_per_tile + k * CHUNK, CHUNK)])
        plsc.subcore_barrier()

        lane = lax.iota(jnp.int32, 16)

        def issue_idx(c, b):
            base = edge_base + c * CHUNK
            pltpu.async_copy(src_hbm.at[pl.ds(base, CHUNK)], src_v.at[b],
                             semi[b])
            pltpu.async_copy(dst_hbm.at[pl.ds(base, CHUNK)], dst_v.at[b],
                             semi[b])

        def wait_idx(b):
            pltpu.make_async_copy(src_hbm.at[pl.ds(0, CHUNK)], src_v.at[b],
                                  semi[b]).wait()
            pltpu.make_async_copy(dst_hbm.at[pl.ds(0, CHUNK)], dst_v.at[b],
                                  semi[b]).wait()

        def issue_gather(b):
            pltpu.async_copy(xl_hbm.at[src_v.at[b]], xl_buf.at[b], semxl[b])
            pltpu.async_copy(xr_hbm.at[dst_v.at[b]], xr_buf.at[b], semxr[b])

        def wait_gather(b):
            pltpu.make_async_copy(xl_hbm.at[src_v.at[b]], xl_buf.at[b],
                                  semxl[b]).wait()
            pltpu.make_async_copy(xr_hbm.at[dst_v.at[b]], xr_buf.at[b],
                                  semxr[b]).wait()

        def compute(b):
            xb = xl_buf.at[b]
            rb = xr_buf.at[b]
            wb = w_buf.at[b]

            def group_body(g, gcarry):
                eidx = g * 16 + lane
                for h in range(H):
                    acc = jnp.zeros((16,), jnp.float32)
                    for c2i in range(C):
                        c = h * C + c2i
                        cvec = jnp.full((16,), c, jnp.int32)
                        xlc = plsc.load_gather(xb, [eidx, cvec])
                        xrc = plsc.load_gather(rb, [eidx, cvec])
                        m = xlc + xrc
                        m = jnp.maximum(m, 0.2 * m)
                        acc = acc + m * att_v[pl.ds(c, 16)][0]
                    ex = jnp.exp(acc)
                    plsc.store_scatter(
                        wb, [eidx, jnp.full((16,), HC + h, jnp.int32)], ex)
                    for c2i in range(C):
                        c = h * C + c2i
                        cvec = jnp.full((16,), c, jnp.int32)
                        xlc = plsc.load_gather(xb, [eidx, cvec])
                        plsc.store_scatter(wb, [eidx, cvec], xlc * ex)
                return gcarry
            lax.fori_loop(0, GROUPS, group_body, 0)

        def scatter_out(b):
            pltpu.sync_copy(w_buf.at[b], num_shared.at[dst_v.at[b]],
                            add=True)

        def half_iter(c, b):
            issue_gather(1 - b)
            wait_gather(b)
            compute(b)
            scatter_out(b)
            @pl.when(c + 2 < NCHUNK)
            def _():
                issue_idx(c + 2, b)

        issue_idx(0, 0)
        wait_idx(0)
        issue_gather(0)
        issue_idx(1, 1)

        def pair_body(i, carry):
            c0 = 2 * i
            wait_idx(1)
            half_iter(c0, 0)
            wait_idx(0)
            half_iter(c0 + 1, 1)
            return carry
        lax.fori_loop(0, NCHUNK // 2, pair_body, 0)

        wait_gather(0)
        compute(0)
        scatter_out(0)

        plsc.subcore_barrier()
        off = cid * N_PAD + sid * rows_per_tile
        pltpu.sync_copy(num_shared.at[pl.ds(sid * rows_per_tile,
                                            rows_per_tile)],
                        out_hbm.at[pl.ds(off, rows_per_tile)])

    return edge_pass


W1 = 136
W2 = 24

_sc_layer1 = _sc_edge_pass(4, 32, W1)
_sc_layer2 = _sc_edge_pass(1, 16, W2)


def _mm2_body(x_ref, wa_ref, wb_ref, oa_ref, ob_ref):
    x = x_ref[...]
    oa_ref[...] = jnp.dot(x, wa_ref[...], preferred_element_type=jnp.float32)
    ob_ref[...] = jnp.dot(x, wb_ref[...], preferred_element_type=jnp.float32)


def _mm2(x, wa_t, wb_t):
    n, d = x.shape
    k = wa_t.shape[1]
    bn = 1000 if n == N else 1024
    return pl.pallas_call(
        _mm2_body,
        grid=(n // bn,),
        in_specs=[
            pl.BlockSpec((bn, d), lambda i: (i, 0)),
            pl.BlockSpec((d, k), lambda i: (0, 0)),
            pl.BlockSpec((d, k), lambda i: (0, 0)),
        ],
        out_specs=[
            pl.BlockSpec((bn, k), lambda i: (i, 0)),
            pl.BlockSpec((bn, k), lambda i: (i, 0)),
        ],
        out_shape=[
            jax.ShapeDtypeStruct((n, k), jnp.float32),
            jax.ShapeDtypeStruct((n, k), jnp.float32),
        ],
    )(x, wa_t, wb_t)


def _combine1_body(s0_ref, s1_ref, sel_ref, b1_ref, wl_ref, wr_ref,
                   xl_ref, xr_ref):
    s0 = s0_ref[...]
    s1 = s1_ref[...]
    num = s0[:, :128] + s1[:, :128]
    den4 = s0[:, 128:132] + s1[:, 128:132]
    den = jnp.dot(den4, sel_ref[...], preferred_element_type=jnp.float32)
    h = jnp.maximum(num / (den + 1e-16) + b1_ref[...], 0.0)
    xl_ref[...] = jnp.dot(h, wl_ref[...], preferred_element_type=jnp.float32)
    xr_ref[...] = jnp.dot(h, wr_ref[...], preferred_element_type=jnp.float32)


def _combine1(s, sel, b1, wl2_t, wr2_t):
    bn = 1024
    k = wl2_t.shape[1]
    return pl.pallas_call(
        _combine1_body,
        grid=(N_PAD // bn,),
        in_specs=[
            pl.BlockSpec((bn, W1), lambda i: (i, 0)),
            pl.BlockSpec((bn, W1), lambda i: (i + N_PAD // bn, 0)),
            pl.BlockSpec((4, 128), lambda i: (0, 0)),
            pl.BlockSpec((1, 128), lambda i: (0, 0)),
            pl.BlockSpec((128, k), lambda i: (0, 0)),
            pl.BlockSpec((128, k), lambda i: (0, 0)),
        ],
        out_specs=[
            pl.BlockSpec((bn, k), lambda i: (i, 0)),
            pl.BlockSpec((bn, k), lambda i: (i, 0)),
        ],
        out_shape=[
            jax.ShapeDtypeStruct((N_PAD, k), jnp.float32),
            jax.ShapeDtypeStruct((N_PAD, k), jnp.float32),
        ],
    )(s, s, sel, b1.reshape(1, 128), wl2_t, wr2_t)


def _combine2_body(s0_ref, s1_ref, b2_ref, o_ref):
    s0 = s0_ref[...]
    s1 = s1_ref[...]
    num = s0[:, :16] + s1[:, :16]
    den = s0[:, 16:17] + s1[:, 16:17]
    o_ref[...] = num / (den + 1e-16) + b2_ref[...]


def _combine2(s, b2):
    bn = 1024
    return pl.pallas_call(
        _combine2_body,
        grid=(N_PAD // bn,),
        in_specs=[
            pl.BlockSpec((bn, W2), lambda i: (i, 0)),
            pl.BlockSpec((bn, W2), lambda i: (i + N_PAD // bn, 0)),
            pl.BlockSpec((1, 16), lambda i: (0, 0)),
        ],
        out_specs=pl.BlockSpec((bn, 16), lambda i: (i, 0)),
        out_shape=jax.ShapeDtypeStruct((N_PAD, 16), jnp.float32),
    )(s, s, b2.reshape(1, 16))


def kernel(x, edge_index, Wl1, Wr1, att1, b1, Wl2, Wr2, att2, b2):
    src = edge_index[0]
    dst = edge_index[1]
    sel = jnp.repeat(jnp.eye(4, dtype=jnp.float32), 32, axis=1)
    att1f = jnp.pad(att1.reshape(-1), (0, 16))
    att2f = jnp.pad(att2.reshape(-1), (0, 16))

    wl1t = jnp.pad(Wl1.T, ((0, 0), (0, W1 - 128)))
    wr1t = jnp.pad(Wr1.T, ((0, 0), (0, W1 - 128)))
    wl2t = jnp.pad(Wl2.T, ((0, 0), (0, W2 - 16)))
    wr2t = jnp.pad(Wr2.T, ((0, 0), (0, W2 - 16)))

    xl1, xr1 = _mm2(x, wl1t, wr1t)
    s1 = _sc_layer1(xl1, xr1, att1f, src, dst)
    xl2, xr2 = _combine1(s1, sel, b1, wl2t, wr2t)
    s2 = _sc_layer2(xl2, xr2, att2f, src, dst)
    out = _combine2(s2, b2)
    return out[:N]


# speedup vs baseline: 21.4731x; 1.0946x over previous
import functools

import jax
import jax.numpy as jnp
from jax import lax
from jax.experimental import pallas as pl
from jax.experimental.pallas import tpu as pltpu
from jax.experimental.pallas import tpu_sc as plsc

N = 10000
E = 320000
N_PAD = 10240
NUM_TILES = 32
EDGES_PER_TILE = E // NUM_TILES
CHUNK = 16
NCHUNK = EDGES_PER_TILE // CHUNK
GROUPS = CHUNK // 16


def _sc_edge_pass(H, C, W):
    HC = H * C
    zero_starts = sorted(set(list(range(0, W - 15, 16)) + [W - 16]))
    mesh = plsc.VectorSubcoreMesh(core_axis_name="c", subcore_axis_name="s")

    @functools.partial(
        pl.kernel,
        mesh=mesh,
        out_type=jax.ShapeDtypeStruct((2 * N_PAD, W), jnp.float32),
        compiler_params=pltpu.CompilerParams(needs_layout_passes=False,
                                             use_tc_tiling_on_sc=False),
        scratch_types=[
            pltpu.VMEM((HC + 16,), jnp.float32),
            pltpu.VMEM((HC, 16), jnp.float32),
            pltpu.VMEM((2, CHUNK), jnp.int32),
            pltpu.VMEM((2, CHUNK), jnp.int32),
            pltpu.VMEM((2, CHUNK, W), jnp.float32),
            pltpu.VMEM((2, CHUNK, W), jnp.float32),
            pltpu.VMEM((2, CHUNK, W), jnp.float32),
            pltpu.VMEM_SHARED((N_PAD, W), jnp.float32),
            pltpu.SemaphoreType.DMA,
            pltpu.SemaphoreType.DMA,
            pltpu.SemaphoreType.DMA,
            pltpu.SemaphoreType.DMA,
            pltpu.SemaphoreType.DMA,
            pltpu.SemaphoreType.DMA,
        ],
    )
    def edge_pass(xl_hbm, xr_hbm, att_hbm, src_hbm, dst_hbm, out_hbm,
                  att_v, att_rot, src_v, dst_v, xl_buf, xr_buf, w_buf,
                  num_shared, semi_a, semi_b, semxl_a, semxl_b, semxr_a,
                  semxr_b):
        cid = lax.axis_index("c")
        sid = lax.axis_index("s")
        wid = cid * 16 + sid
        edge_base = wid * EDGES_PER_TILE
        semi = (semi_a, semi_b)
        semxl = (semxl_a, semxl_b)
        semxr = (semxr_a, semxr_b)

        pltpu.sync_copy(att_hbm, att_v)

        lane0 = lax.iota(jnp.int32, 16)
        for h0 in range(H):
            for c20 in range(C):
                rotv = h0 * C + ((c20 + lane0) & (C - 1))
                att_rot[h0 * C + c20, :] = plsc.load_gather(att_v, [rotv])

        for b in range(2):
            def zero_body(j, carry, _b=b):
                for st in zero_starts:
                    w_buf[_b, j, pl.ds(st, 16)] = jnp.zeros((16,),
                                                            jnp.float32)
                return carry
            lax.fori_loop(0, CHUNK, zero_body, 0)
        rows_per_tile = N_PAD // 16
        for k in range(rows_per_tile // CHUNK):
            pltpu.sync_copy(
                w_buf.at[0],
                num_shared.at[pl.ds(sid * rows_per_tile + k * CHUNK, CHUNK)])
        plsc.subcore_barrier()

        lane = lax.iota(jnp.int32, 16)

        def issue_idx(c, b):
            base = edge_base + c * CHUNK
            pltpu.async_copy(src_hbm.at[pl.ds(base, CHUNK)], src_v.at[b],
                             semi[b])
            pltpu.async_copy(dst_hbm.at[pl.ds(base, CHUNK)], dst_v.at[b],
                             semi[b])

        def wait_idx(b):
            pltpu.make_async_copy(src_hbm.at[pl.ds(0, CHUNK)], src_v.at[b],
                                  semi[b]).wait()
            pltpu.make_async_copy(dst_hbm.at[pl.ds(0, CHUNK)], dst_v.at[b],
                                  semi[b]).wait()

        def issue_gather(b):
            pltpu.async_copy(xl_hbm.at[src_v.at[b]], xl_buf.at[b], semxl[b])
            pltpu.async_copy(xr_hbm.at[dst_v.at[b]], xr_buf.at[b], semxr[b])

        def wait_gather(b):
            pltpu.make_async_copy(xl_hbm.at[src_v.at[b]], xl_buf.at[b],
                                  semxl[b]).wait()
            pltpu.make_async_copy(xr_hbm.at[dst_v.at[b]], xr_buf.at[b],
                                  semxr[b]).wait()

        def compute(b):
            xb = xl_buf.at[b]
            rb = xr_buf.at[b]
            wb = w_buf.at[b]

            def group_body(g, gcarry):
                eidx = g * 16 + lane
                for h in range(H):
                    acc = jnp.zeros((16,), jnp.float32)
                    for c2i in range(C):
                        c = h * C + c2i
                        cvec = h * C + ((c2i + lane) & (C - 1))
                        xlc = plsc.load_gather(xb, [eidx, cvec])
                        xrc = plsc.load_gather(rb, [eidx, cvec])
                        m = xlc + xrc
                        m = jnp.maximum(m, 0.2 * m)
                        acc = acc + m * att_rot[c, :]
                    ex = jnp.exp(acc)
                    plsc.store_scatter(
                        wb, [eidx, jnp.full((16,), HC + h, jnp.int32)], ex)
                    for c2i in range(C):
                        c = h * C + c2i
                        cvec = h * C + ((c2i + lane) & (C - 1))
                        xlc = plsc.load_gather(xb, [eidx, cvec])
                        plsc.store_scatter(wb, [eidx, cvec], xlc * ex)
                return gcarry
            lax.fori_loop(0, GROUPS, group_body, 0)

        def scatter_out(b):
            pltpu.sync_copy(w_buf.at[b], num_shared.at[dst_v.at[b]],
                            add=True)

        def half_iter(c, b):
            issue_gather(1 - b)
            wait_gather(b)
            compute(b)
            scatter_out(b)
            @pl.when(c + 2 < NCHUNK)
            def _():
                issue_idx(c + 2, b)

        issue_idx(0, 0)
        wait_idx(0)
        issue_gather(0)
        issue_idx(1, 1)

        def pair_body(i, carry):
            c0 = 2 * i
            wait_idx(1)
            half_iter(c0, 0)
            wait_idx(0)
            half_iter(c0 + 1, 1)
            return carry
        lax.fori_loop(0, NCHUNK // 2, pair_body, 0)

        wait_gather(0)
        compute(0)
        scatter_out(0)

        plsc.subcore_barrier()
        off = cid * N_PAD + sid * rows_per_tile
        pltpu.sync_copy(num_shared.at[pl.ds(sid * rows_per_tile,
                                            rows_per_tile)],
                        out_hbm.at[pl.ds(off, rows_per_tile)])

    return edge_pass


W1 = 136
W2 = 24

_sc_layer1 = _sc_edge_pass(4, 32, W1)
_sc_layer2 = _sc_edge_pass(1, 16, W2)


def _mm2_body(x_ref, wa_ref, wb_ref, oa_ref, ob_ref):
    x = x_ref[...]
    oa_ref[...] = jnp.dot(x, wa_ref[...], preferred_element_type=jnp.float32)
    ob_ref[...] = jnp.dot(x, wb_ref[...], preferred_element_type=jnp.float32)


def _mm2(x, wa_t, wb_t):
    n, d = x.shape
    k = wa_t.shape[1]
    bn = 1000 if n == N else 1024
    return pl.pallas_call(
        _mm2_body,
        grid=(n // bn,),
        in_specs=[
            pl.BlockSpec((bn, d), lambda i: (i, 0)),
            pl.BlockSpec((d, k), lambda i: (0, 0)),
            pl.BlockSpec((d, k), lambda i: (0, 0)),
        ],
        out_specs=[
            pl.BlockSpec((bn, k), lambda i: (i, 0)),
            pl.BlockSpec((bn, k), lambda i: (i, 0)),
        ],
        out_shape=[
            jax.ShapeDtypeStruct((n, k), jnp.float32),
            jax.ShapeDtypeStruct((n, k), jnp.float32),
        ],
    )(x, wa_t, wb_t)


def _combine1_body(s0_ref, s1_ref, sel_ref, b1_ref, wl_ref, wr_ref,
                   xl_ref, xr_ref):
    s0 = s0_ref[...]
    s1 = s1_ref[...]
    num = s0[:, :128] + s1[:, :128]
    den4 = s0[:, 128:132] + s1[:, 128:132]
    den = jnp.dot(den4, sel_ref[...], preferred_element_type=jnp.float32)
    h = jnp.maximum(num / (den + 1e-16) + b1_ref[...], 0.0)
    xl_ref[...] = jnp.dot(h, wl_ref[...], preferred_element_type=jnp.float32)
    xr_ref[...] = jnp.dot(h, wr_ref[...], preferred_element_type=jnp.float32)


def _combine1(s, sel, b1, wl2_t, wr2_t):
    bn = 1024
    k = wl2_t.shape[1]
    return pl.pallas_call(
        _combine1_body,
        grid=(N_PAD // bn,),
        in_specs=[
            pl.BlockSpec((bn, W1), lambda i: (i, 0)),
            pl.BlockSpec((bn, W1), lambda i: (i + N_PAD // bn, 0)),
            pl.BlockSpec((4, 128), lambda i: (0, 0)),
            pl.BlockSpec((1, 128), lambda i: (0, 0)),
            pl.BlockSpec((128, k), lambda i: (0, 0)),
            pl.BlockSpec((128, k), lambda i: (0, 0)),
        ],
        out_specs=[
            pl.BlockSpec((bn, k), lambda i: (i, 0)),
            pl.BlockSpec((bn, k), lambda i: (i, 0)),
        ],
        out_shape=[
            jax.ShapeDtypeStruct((N_PAD, k), jnp.float32),
            jax.ShapeDtypeStruct((N_PAD, k), jnp.float32),
        ],
    )(s, s, sel, b1.reshape(1, 128), wl2_t, wr2_t)


def _combine2_body(s0_ref, s1_ref, b2_ref, o_ref):
    s0 = s0_ref[...]
    s1 = s1_ref[...]
    num = s0[:, :16] + s1[:, :16]
    den = s0[:, 16:17] + s1[:, 16:17]
    o_ref[...] = num / (den + 1e-16) + b2_ref[...]


def _combine2(s, b2):
    bn = 1024
    return pl.pallas_call(
        _combine2_body,
        grid=(N_PAD // bn,),
        in_specs=[
            pl.BlockSpec((bn, W2), lambda i: (i, 0)),
            pl.BlockSpec((bn, W2), lambda i: (i + N_PAD // bn, 0)),
            pl.BlockSpec((1, 16), lambda i: (0, 0)),
        ],
        out_specs=pl.BlockSpec((bn, 16), lambda i: (i, 0)),
        out_shape=jax.ShapeDtypeStruct((N_PAD, 16), jnp.float32),
    )(s, s, b2.reshape(1, 16))


def kernel(x, edge_index, Wl1, Wr1, att1, b1, Wl2, Wr2, att2, b2):
    src = edge_index[0]
    dst = edge_index[1]
    sel = jnp.repeat(jnp.eye(4, dtype=jnp.float32), 32, axis=1)
    att1f = jnp.pad(att1.reshape(-1), (0, 16))
    att2f = jnp.pad(att2.reshape(-1), (0, 16))

    wl1t = jnp.pad(Wl1.T, ((0, 0), (0, W1 - 128)))
    wr1t = jnp.pad(Wr1.T, ((0, 0), (0, W1 - 128)))
    wl2t = jnp.pad(Wl2.T, ((0, 0), (0, W2 - 16)))
    wr2t = jnp.pad(Wr2.T, ((0, 0), (0, W2 - 16)))

    xl1, xr1 = _mm2(x, wl1t, wr1t)
    s1 = _sc_layer1(xl1, xr1, att1f, src, dst)
    xl2, xr2 = _combine1(s1, sel, b1, wl2t, wr2t)
    s2 = _sc_layer2(xl2, xr2, att2f, src, dst)
    out = _combine2(s2, b2)
    return out[:N]
